# baseline (speedup 1.0000x reference)
import math

import jax
import jax.numpy as jnp
import numpy as np
from jax.experimental import pallas as pl
from jax.experimental.pallas import tpu as pltpu

_NUM_EXPERTS = 8
_TOP_K = 2
_EMBED = 768
_HEADS = 12
_SEQ = 256
_BATCH = 16
_HEAD_DIM = _EMBED // _HEADS
_PAIRS = _BATCH * _TOP_K
_EPS = float(np.finfo(np.float64).eps)


def _gating_body(q_ref, wg_ref, wn_ref, noise_ref, loss_ref, ob_ref, oe_ref):
    B, N = _BATCH, _NUM_EXPERTS
    qsum = jnp.sum(q_ref[...], axis=0)
    clean = jnp.dot(qsum, wg_ref[...], preferred_element_type=jnp.float32)
    rawn = jnp.dot(qsum, wn_ref[...], preferred_element_type=jnp.float32)
    std = jax.nn.softplus(rawn) + 0.01
    noisy = clean + noise_ref[...] * std
    iota = jax.lax.broadcasted_iota(jnp.int32, (B, N), 1)
    masked = noisy
    vals, idxs = [], []
    for _ in range(_TOP_K + 1):
        v = jnp.max(masked, axis=1, keepdims=True)
        sel = jnp.min(jnp.where(masked >= v, iota, N), axis=1, keepdims=True)
        vals.append(v)
        idxs.append(sel)
        masked = jnp.where(iota == sel, -jnp.inf, masked)
    v0, v1, v2 = vals
    i0, i1 = idxs[0], idxs[1]
    e1 = jnp.exp(v1 - v0)
    g0 = 1.0 / (1.0 + e1)
    g1 = e1 / (1.0 + e1)
    gates = jnp.where(iota == i0, g0, 0.0) + jnp.where(iota == i1, g1, 0.0)
    importance = jnp.sum(gates, axis=0, keepdims=True)
    inv_sqrt2 = 0.7071067811865476
    is_in = noisy > v2
    prob_in = 0.5 * (1.0 + jax.lax.erf((clean - v2) / std * inv_sqrt2))
    prob_out = 0.5 * (1.0 + jax.lax.erf((clean - v1) / std * inv_sqrt2))
    load = jnp.sum(jnp.where(is_in, prob_in, prob_out), axis=0, keepdims=True)

    def cv2(x):
        mu = jnp.mean(x)
        var = jnp.sum((x - mu) ** 2) / (N - 1)
        return var / (mu * mu + 1e-10)

    loss_ref[...] = ((cv2(importance) + cv2(load)) * 0.01).reshape(1, 1)

    P = _PAIRS
    eye = jnp.where(
        jax.lax.broadcasted_iota(jnp.int32, (P, P), 0)
        == jax.lax.broadcasted_iota(jnp.int32, (P, P), 1), 1.0, 0.0)
    pcol = jax.lax.broadcasted_iota(jnp.int32, (P, 1), 0)
    bcol = pcol // _TOP_K
    kcol = pcol - bcol * _TOP_K
    bmat = jax.lax.broadcasted_iota(jnp.int32, (P, B), 1)
    w0 = jnp.where((bcol == bmat) & (kcol == 0), 1.0, 0.0)
    w1 = jnp.where((bcol == bmat) & (kcol == 1), 1.0, 0.0)
    ecol = (jnp.dot(w0, i0.astype(jnp.float32),
                    preferred_element_type=jnp.float32)
            + jnp.dot(w1, i1.astype(jnp.float32),
                      preferred_element_type=jnp.float32))
    key_col = ecol * P + pcol.astype(jnp.float32)

    def trans(c):
        return jax.lax.dot_general(c, eye, (((0,), (0,)), ((), ())),
                                   preferred_element_type=jnp.float32)

    rank_col = jnp.sum(jnp.where(trans(key_col) < key_col, 1.0, 0.0),
                       axis=1, keepdims=True)
    m = jnp.where(trans(rank_col) == pcol.astype(jnp.float32), 1.0, 0.0)
    ob = jnp.dot(m, bcol.astype(jnp.float32), preferred_element_type=jnp.float32)
    oe = jnp.dot(m, ecol, preferred_element_type=jnp.float32)
    ob_ref[...] = ob.astype(jnp.int32)
    oe_ref[...] = oe.astype(jnp.int32)


def _moe_body(ob_ref, oe_ref, q_ref, k_ref, v_ref, wi_ref, bi_ref, wo_ref,
              bo_ref, out_ref, w_ref):
    E, H, dh = _EMBED, _HEADS, _HEAD_DIM
    i = pl.program_id(0)
    b = ob_ref[i]
    q = q_ref[0]
    kk = k_ref[0]
    vv = v_ref[0]
    Wi = wi_ref[0]
    bi = bi_ref[0]
    Wo = wo_ref[0]
    bo = bo_ref[0]

    def nt(a, b):
        return jax.lax.dot_general(a, b, (((1,), (1,)), ((), ())),
                                   preferred_element_type=jnp.float32)

    bf = jnp.bfloat16
    qp = (nt(q, Wi[:E]) + bi[:, :E]).astype(bf)
    kp = (nt(kk, Wi[E:2 * E]) + bi[:, E:2 * E]).astype(bf)
    vp = (nt(vv, Wi[2 * E:]) + bi[:, 2 * E:]).astype(bf)
    scale = 1.0 / math.sqrt(dh)
    attn_sum = jnp.zeros((_SEQ, _SEQ), jnp.float32)
    parts = []
    for h in range(H):
        sl = slice(h * dh, (h + 1) * dh)
        attn = jax.nn.softmax(nt(qp[:, sl], kp[:, sl]) * scale, axis=-1)
        attn_sum = attn_sum + attn
        parts.append(jnp.dot(attn.astype(bf), vp[:, sl],
                             preferred_element_type=jnp.float32).astype(bf))
    out = nt(jnp.concatenate(parts, axis=1), Wo) + bo
    exp_out = jnp.exp(out)
    exp_w = jnp.exp(attn_sum * (1.0 / H))

    @pl.when(i == 0)
    def _():
        out_ref[...] = jnp.zeros_like(out_ref)
        w_ref[...] = jnp.zeros_like(w_ref)

    out_ref[:, b, :] = out_ref[:, b, :] + exp_out
    w_ref[b] = w_ref[b] + exp_w

    @pl.when(i == _PAIRS - 1)
    def _():
        tot = out_ref[...]
        out_ref[...] = jnp.log(jnp.where(tot == 0.0, _EPS, tot))
        totw = w_ref[...]
        w_ref[...] = jnp.log(jnp.where(totw == 0.0, _EPS, totw))


def kernel(query, key, value, w_gate, w_noise, in_proj_w, in_proj_b,
           out_proj_w, out_proj_b):
    S, B, E = _SEQ, _BATCH, _EMBED
    noise = jax.random.normal(jax.random.key(1234), (B, _NUM_EXPERTS),
                              dtype=jnp.float32)
    loss2, ob2, oe2 = pl.pallas_call(
        _gating_body,
        out_shape=(
            jax.ShapeDtypeStruct((1, 1), jnp.float32),
            jax.ShapeDtypeStruct((_PAIRS, 1), jnp.int32),
            jax.ShapeDtypeStruct((_PAIRS, 1), jnp.int32),
        ),
    )(query, w_gate, w_noise, noise)
    loss = loss2[0, 0]
    ob = ob2.reshape(_PAIRS)
    oe = oe2.reshape(_PAIRS)

    bi3 = in_proj_b.reshape(_NUM_EXPERTS, 1, 3 * E)
    bo3 = out_proj_b.reshape(_NUM_EXPERTS, 1, E)
    qb = jnp.swapaxes(query, 0, 1).astype(jnp.bfloat16)
    kb = jnp.swapaxes(key, 0, 1).astype(jnp.bfloat16)
    vb = jnp.swapaxes(value, 0, 1).astype(jnp.bfloat16)
    wib = in_proj_w.astype(jnp.bfloat16)
    wob = out_proj_w.astype(jnp.bfloat16)

    grid_spec = pltpu.PrefetchScalarGridSpec(
        num_scalar_prefetch=2,
        grid=(_PAIRS,),
        in_specs=[
            pl.BlockSpec((1, S, E), lambda i, ob, oe: (ob[i], 0, 0)),
            pl.BlockSpec((1, S, E), lambda i, ob, oe: (ob[i], 0, 0)),
            pl.BlockSpec((1, S, E), lambda i, ob, oe: (ob[i], 0, 0)),
            pl.BlockSpec((1, 3 * E, E), lambda i, ob, oe: (oe[i], 0, 0)),
            pl.BlockSpec((1, 1, 3 * E), lambda i, ob, oe: (oe[i], 0, 0)),
            pl.BlockSpec((1, E, E), lambda i, ob, oe: (oe[i], 0, 0)),
            pl.BlockSpec((1, 1, E), lambda i, ob, oe: (oe[i], 0, 0)),
        ],
        out_specs=[
            pl.BlockSpec((S, B, E), lambda i, ob, oe: (0, 0, 0)),
            pl.BlockSpec((B, S, S), lambda i, ob, oe: (0, 0, 0)),
        ],
    )
    out_log, w_log = pl.pallas_call(
        _moe_body,
        grid_spec=grid_spec,
        out_shape=(
            jax.ShapeDtypeStruct((S, B, E), jnp.float32),
            jax.ShapeDtypeStruct((B, S, S), jnp.float32),
        ),
        compiler_params=pltpu.CompilerParams(
            vmem_limit_bytes=120 * 1024 * 1024,
        ),
    )(ob, oe, qb, kb, vb, wib, bi3, wob, bo3)
    return (out_log, loss, w_log)


# speedup vs baseline: 1.1248x; 1.1248x over previous
import math

import jax
import jax.numpy as jnp
import numpy as np
from jax.experimental import pallas as pl
from jax.experimental.pallas import tpu as pltpu

_NUM_EXPERTS = 8
_TOP_K = 2
_EMBED = 768
_HEADS = 12
_SEQ = 256
_BATCH = 16
_HEAD_DIM = _EMBED // _HEADS
_PAIRS = _BATCH * _TOP_K
_EPS = float(np.finfo(np.float64).eps)


def _gating_body(q_ref, wg_ref, wn_ref, noise_ref, loss_ref, ob_ref, oe_ref):
    B, N = _BATCH, _NUM_EXPERTS
    qsum = jnp.sum(q_ref[...], axis=0)
    clean = jnp.dot(qsum, wg_ref[...], preferred_element_type=jnp.float32)
    rawn = jnp.dot(qsum, wn_ref[...], preferred_element_type=jnp.float32)
    std = jax.nn.softplus(rawn) + 0.01
    noisy = clean + noise_ref[...] * std
    iota = jax.lax.broadcasted_iota(jnp.int32, (B, N), 1)
    masked = noisy
    vals, idxs = [], []
    for _ in range(_TOP_K + 1):
        v = jnp.max(masked, axis=1, keepdims=True)
        sel = jnp.min(jnp.where(masked >= v, iota, N), axis=1, keepdims=True)
        vals.append(v)
        idxs.append(sel)
        masked = jnp.where(iota == sel, -jnp.inf, masked)
    v0, v1, v2 = vals
    i0, i1 = idxs[0], idxs[1]
    e1 = jnp.exp(v1 - v0)
    g0 = 1.0 / (1.0 + e1)
    g1 = e1 / (1.0 + e1)
    gates = jnp.where(iota == i0, g0, 0.0) + jnp.where(iota == i1, g1, 0.0)
    importance = jnp.sum(gates, axis=0, keepdims=True)
    inv_sqrt2 = 0.7071067811865476
    is_in = noisy > v2
    prob_in = 0.5 * (1.0 + jax.lax.erf((clean - v2) / std * inv_sqrt2))
    prob_out = 0.5 * (1.0 + jax.lax.erf((clean - v1) / std * inv_sqrt2))
    load = jnp.sum(jnp.where(is_in, prob_in, prob_out), axis=0, keepdims=True)

    def cv2(x):
        mu = jnp.mean(x)
        var = jnp.sum((x - mu) ** 2) / (N - 1)
        return var / (mu * mu + 1e-10)

    loss_ref[...] = ((cv2(importance) + cv2(load)) * 0.01).reshape(1, 1)

    P = _PAIRS
    eye = jnp.where(
        jax.lax.broadcasted_iota(jnp.int32, (P, P), 0)
        == jax.lax.broadcasted_iota(jnp.int32, (P, P), 1), 1.0, 0.0)
    pcol = jax.lax.broadcasted_iota(jnp.int32, (P, 1), 0)
    bcol = pcol // _TOP_K
    kcol = pcol - bcol * _TOP_K
    bmat = jax.lax.broadcasted_iota(jnp.int32, (P, B), 1)
    w0 = jnp.where((bcol == bmat) & (kcol == 0), 1.0, 0.0)
    w1 = jnp.where((bcol == bmat) & (kcol == 1), 1.0, 0.0)
    ecol = (jnp.dot(w0, i0.astype(jnp.float32),
                    preferred_element_type=jnp.float32)
            + jnp.dot(w1, i1.astype(jnp.float32),
                      preferred_element_type=jnp.float32))
    key_col = ecol * P + pcol.astype(jnp.float32)

    def trans(c):
        return jax.lax.dot_general(c, eye, (((0,), (0,)), ((), ())),
                                   preferred_element_type=jnp.float32)

    rank_col = jnp.sum(jnp.where(trans(key_col) < key_col, 1.0, 0.0),
                       axis=1, keepdims=True)
    m = jnp.where(trans(rank_col) == pcol.astype(jnp.float32), 1.0, 0.0)
    ob = jnp.dot(m, bcol.astype(jnp.float32), preferred_element_type=jnp.float32)
    oe = jnp.dot(m, ecol, preferred_element_type=jnp.float32)
    ob_ref[...] = ob.astype(jnp.int32)
    oe_ref[...] = oe.astype(jnp.int32)


def _moe_body(ob_ref, oe_ref, q_ref, k_ref, v_ref, wi_ref, bi_ref, wo_ref,
              bo_ref, out_ref, w_ref, wi_s, wo_s):
    E, H, dh = _EMBED, _HEADS, _HEAD_DIM
    bf = jnp.bfloat16
    i = pl.program_id(0)
    b = ob_ref[i]
    q = q_ref[0]
    kk = k_ref[0]
    vv = v_ref[0]
    bi = bi_ref[0]
    bo = bo_ref[0]

    changed = jnp.logical_or(i == 0,
                             oe_ref[i] != oe_ref[jnp.maximum(i - 1, 0)])

    @pl.when(changed)
    def _():
        wi_s[...] = wi_ref[0].astype(bf)
        wo_s[...] = wo_ref[0].astype(bf)

    Wi = wi_s[...]
    Wo = wo_s[...]

    def nt(a, b):
        return jax.lax.dot_general(a, b, (((1,), (1,)), ((), ())),
                                   preferred_element_type=jnp.float32)

    qp = (nt(q, Wi[:E]) + bi[:, :E]).astype(bf)
    kp = (nt(kk, Wi[E:2 * E]) + bi[:, E:2 * E]).astype(bf)
    vp = (nt(vv, Wi[2 * E:]) + bi[:, 2 * E:]).astype(bf)
    scale = 1.0 / math.sqrt(dh)
    attn_sum = jnp.zeros((_SEQ, _SEQ), jnp.float32)
    parts = []
    for h in range(H):
        sl = slice(h * dh, (h + 1) * dh)
        attn = jax.nn.softmax(nt(qp[:, sl], kp[:, sl]) * scale, axis=-1)
        attn_sum = attn_sum + attn
        parts.append(jnp.dot(attn.astype(bf), vp[:, sl],
                             preferred_element_type=jnp.float32).astype(bf))
    out = nt(jnp.concatenate(parts, axis=1), Wo) + bo
    exp_out = jnp.exp(out)
    exp_w = jnp.exp(attn_sum * (1.0 / H))

    @pl.when(i == 0)
    def _():
        out_ref[...] = jnp.zeros_like(out_ref)
        w_ref[...] = jnp.zeros_like(w_ref)

    out_ref[:, b, :] = out_ref[:, b, :] + exp_out
    w_ref[b] = w_ref[b] + exp_w

    @pl.when(i == _PAIRS - 1)
    def _():
        tot = out_ref[...]
        out_ref[...] = jnp.log(jnp.where(tot == 0.0, _EPS, tot))
        totw = w_ref[...]
        w_ref[...] = jnp.log(jnp.where(totw == 0.0, _EPS, totw))


def kernel(query, key, value, w_gate, w_noise, in_proj_w, in_proj_b,
           out_proj_w, out_proj_b):
    S, B, E = _SEQ, _BATCH, _EMBED
    noise = jax.random.normal(jax.random.key(1234), (B, _NUM_EXPERTS),
                              dtype=jnp.float32)
    loss2, ob2, oe2 = pl.pallas_call(
        _gating_body,
        out_shape=(
            jax.ShapeDtypeStruct((1, 1), jnp.float32),
            jax.ShapeDtypeStruct((_PAIRS, 1), jnp.int32),
            jax.ShapeDtypeStruct((_PAIRS, 1), jnp.int32),
        ),
    )(query, w_gate, w_noise, noise)
    loss = loss2[0, 0]
    ob = ob2.reshape(_PAIRS)
    oe = oe2.reshape(_PAIRS)

    bi3 = in_proj_b.reshape(_NUM_EXPERTS, 1, 3 * E)
    bo3 = out_proj_b.reshape(_NUM_EXPERTS, 1, E)
    qb = jnp.swapaxes(query, 0, 1).astype(jnp.bfloat16)
    kb = jnp.swapaxes(key, 0, 1).astype(jnp.bfloat16)
    vb = jnp.swapaxes(value, 0, 1).astype(jnp.bfloat16)
    grid_spec = pltpu.PrefetchScalarGridSpec(
        num_scalar_prefetch=2,
        grid=(_PAIRS,),
        scratch_shapes=[
            pltpu.VMEM((3 * E, E), jnp.bfloat16),
            pltpu.VMEM((E, E), jnp.bfloat16),
        ],
        in_specs=[
            pl.BlockSpec((1, S, E), lambda i, ob, oe: (ob[i], 0, 0)),
            pl.BlockSpec((1, S, E), lambda i, ob, oe: (ob[i], 0, 0)),
            pl.BlockSpec((1, S, E), lambda i, ob, oe: (ob[i], 0, 0)),
            pl.BlockSpec((1, 3 * E, E), lambda i, ob, oe: (oe[i], 0, 0)),
            pl.BlockSpec((1, 1, 3 * E), lambda i, ob, oe: (oe[i], 0, 0)),
            pl.BlockSpec((1, E, E), lambda i, ob, oe: (oe[i], 0, 0)),
            pl.BlockSpec((1, 1, E), lambda i, ob, oe: (oe[i], 0, 0)),
        ],
        out_specs=[
            pl.BlockSpec((S, B, E), lambda i, ob, oe: (0, 0, 0)),
            pl.BlockSpec((B, S, S), lambda i, ob, oe: (0, 0, 0)),
        ],
    )
    out_log, w_log = pl.pallas_call(
        _moe_body,
        grid_spec=grid_spec,
        out_shape=(
            jax.ShapeDtypeStruct((S, B, E), jnp.float32),
            jax.ShapeDtypeStruct((B, S, S), jnp.float32),
        ),
        compiler_params=pltpu.CompilerParams(
            vmem_limit_bytes=120 * 1024 * 1024,
        ),
    )(ob, oe, qb, kb, vb, in_proj_w, bi3, out_proj_w, bo3)
    return (out_log, loss, w_log)
